# baseline (speedup 1.0000x reference)
import functools

import jax
import jax.numpy as jnp
from jax import lax
from jax.experimental import pallas as pl
from jax.experimental.pallas import tpu as pltpu
from jax.experimental.pallas import tpu_sc as plsc

_N = 1024
_T = 50
_V = 1000
_F = 10
_TP = 56
_WP = 128

_info = plsc.get_sparse_core_info()
_NC, _NS, _L = _info.num_cores, _info.num_subcores, _info.num_lanes
_NW = _NC * _NS
_NPW = _N // _NW
_SLAB_N = 8
_SLABS = _NPW // _SLAB_N
_SLAB_ROWS = _SLAB_N * _T
_SLAB_X = _SLAB_ROWS * _F
_XW = _NPW * _T * _F

_mesh = plsc.VectorSubcoreMesh(core_axis_name="c", subcore_axis_name="s")


@functools.partial(
    pl.kernel,
    out_type=jax.ShapeDtypeStruct((_N, _TP, _WP), jnp.int32),
    mesh=_mesh,
    scratch_types=[
        pltpu.VMEM((_SLAB_N, _TP, _WP), jnp.int32),
        pltpu.VMEM((_SLAB_N, _TP, _WP), jnp.int32),
        pltpu.VMEM((_XW,), jnp.int32),
        pltpu.SemaphoreType.DMA,
        pltpu.SemaphoreType.DMA,
    ],
    compiler_params=pltpu.CompilerParams(needs_layout_passes=False),
)
def _sc_bitmap(x_hbm, bm_hbm, buf0, buf1, xv, sem0, sem1):
    wid = lax.axis_index("s") * _NC + lax.axis_index("c")
    n0 = wid * _NPW

    zero = jnp.zeros((_L,), jnp.int32)
    one = jnp.ones((_L,), jnp.int32)
    lane = lax.iota(jnp.int32, _L)

    pltpu.sync_copy(x_hbm.at[pl.ds(n0 * _T * _F, _XW)], xv)

    for buf in (buf0, buf1):
        def za(a, carry, buf=buf):
            def zt(t, c):
                buf[a, t, pl.ds(0, _L)] = zero
                buf[a, t, pl.ds(_L, _L)] = zero
                return c

            return lax.fori_loop(0, _T, zt, carry)

        lax.fori_loop(0, _SLAB_N, za, 0)

    def coords(s, f, j):
        r = j * _L + lane
        nl = lax.shift_right_logical(r * 41944, 21)
        t = r - nl * _T
        v = plsc.load_gather(xv, [s * _SLAB_X + r * _F + f])
        w = lax.shift_right_logical(v, 5)
        bit = lax.shift_left(one, jnp.bitwise_and(v, 31))
        return nl, t, w, bit

    def scatter_slab(buf, s):
        for f in range(_F):
            def body(j, carry, f=f):
                nl, t, w, bit = coords(s, f, j)
                old = plsc.load_gather(buf, [nl, t, w])
                plsc.store_scatter(buf, [nl, t, w], jnp.bitwise_or(old, bit))
                return carry

            lax.fori_loop(0, _SLAB_ROWS // _L, body, 0)

    def clean_slab(buf, s):
        for f in range(_F):
            def body(j, carry, f=f):
                nl, t, w, _ = coords(s, f, j)
                plsc.store_scatter(buf, [nl, t, w], zero)
                return carry

            lax.fori_loop(0, _SLAB_ROWS // _L, body, 0)

    def issue_out(buf, sem, s):
        pltpu.async_copy(buf, bm_hbm.at[pl.ds(n0 + s * _SLAB_N, _SLAB_N)], sem)

    def wait_out(buf, sem):
        pltpu.make_async_copy(buf, bm_hbm.at[pl.ds(0, _SLAB_N)], sem).wait()

    scatter_slab(buf0, 0)
    issue_out(buf0, sem0, 0)
    scatter_slab(buf1, 1)
    issue_out(buf1, sem1, 1)
    wait_out(buf0, sem0)
    clean_slab(buf0, 0)
    scatter_slab(buf0, 2)
    issue_out(buf0, sem0, 2)
    wait_out(buf1, sem1)
    clean_slab(buf1, 1)
    scatter_slab(buf1, 3)
    issue_out(buf1, sem1, 3)
    wait_out(buf0, sem0)
    wait_out(buf1, sem1)


_BN = 16
_GRID = _N // _BN


def _tc_expand_body(bm_ref, out_ref):
    bm = bm_ref[...]
    shift = jnp.bitwise_and(
        lax.broadcasted_iota(jnp.int32, (_BN, _T, _WP), 2), 31
    )
    pieces = []
    for c in range(_V // 128 + 1):
        words = []
        for k in range(4):
            w1 = bm[:, : _T, 4 * c + k : 4 * c + k + 1]
            words.append(
                lax.broadcast_in_dim(w1, (_BN, _T, 32), (0, 1, 2))
            )
        piece = jnp.concatenate(words, axis=2)
        bits = jnp.bitwise_and(lax.shift_right_logical(piece, shift), 1)
        pieces.append(bits)
    full = jnp.concatenate(pieces, axis=2)[:, :, : _V]
    out_ref[...] = full.astype(jnp.float32)


_tc_expand = pl.pallas_call(
    _tc_expand_body,
    grid=(_GRID,),
    in_specs=[pl.BlockSpec((_BN, _TP, _WP), lambda i: (i, 0, 0))],
    out_specs=pl.BlockSpec((_BN, _T, _V), lambda i: (i, 0, 0)),
    out_shape=jax.ShapeDtypeStruct((_N, _T, _V), jnp.float32),
)


def kernel(x):
    bm = _sc_bitmap(x.reshape(-1))
    return _tc_expand(bm)


# speedup vs baseline: 1.4711x; 1.4711x over previous
import functools

import jax
import jax.numpy as jnp
from jax import lax
from jax.experimental import pallas as pl
from jax.experimental.pallas import tpu as pltpu
from jax.experimental.pallas import tpu_sc as plsc

_N = 1024
_T = 50
_V = 1000
_F = 10

_info = plsc.get_sparse_core_info()
_NC, _NS, _L = _info.num_cores, _info.num_subcores, _info.num_lanes
_NW = _NC * _NS
_NPW = _N // _NW
_XSLAB = _T * _F
_XW = _NPW * _XSLAB
_SVEC = (_XSLAB + _L - 1) // _L
_ZVEC = (_V + _L - 1) // _L

_mesh = plsc.VectorSubcoreMesh(core_axis_name="c", subcore_axis_name="s")


@functools.partial(
    pl.kernel,
    out_type=jax.ShapeDtypeStruct((_N, _T, _V), jnp.float32),
    mesh=_mesh,
    scratch_types=[
        pltpu.VMEM((_T, _V), jnp.float32),
        pltpu.VMEM((_T, _V), jnp.float32),
        pltpu.VMEM((_XW,), jnp.int32),
        pltpu.SemaphoreType.DMA,
        pltpu.SemaphoreType.DMA,
    ],
    compiler_params=pltpu.CompilerParams(
        needs_layout_passes=False, use_tc_tiling_on_sc=True
    ),
)
def _ten_hot(x_hbm, out_hbm, buf0, buf1, xv, sem0, sem1):
    wid = lax.axis_index("s") * _NC + lax.axis_index("c")
    n0 = wid * _NPW

    zeros = jnp.zeros((_L,), jnp.float32)
    ones = jnp.ones((_L,), jnp.float32)
    lane = lax.iota(jnp.int32, _L)

    pltpu.sync_copy(x_hbm.at[pl.ds(n0 * _XSLAB, _XW)], xv)

    for buf in (buf0, buf1):
        def zrow(t, carry, buf=buf):
            def zcol(j, c):
                off = jnp.minimum(j * _L, _V - _L)
                buf[t, pl.ds(off, _L)] = zeros
                return c

            return lax.fori_loop(0, _ZVEC, zcol, carry)

        lax.fori_loop(0, _T, zrow, 0)

    def scatter_slab(buf, k, val):
        xoff = k * _XSLAB

        def body(i, carry):
            e = jnp.minimum(i * _L, _XSLAB - _L) + lane
            t = lax.shift_right_logical(e * 52429, 19)
            v = xv[pl.ds(jnp.minimum(i * _L, _XSLAB - _L) + xoff, _L)]
            plsc.store_scatter(buf, [t, v], val)
            return carry

        lax.fori_loop(0, _SVEC, body, 0)

    def issue_out(buf, sem, k):
        pltpu.async_copy(buf, out_hbm.at[n0 + k], sem)

    for b, (buf, sem) in enumerate(((buf0, sem0), (buf1, sem1))):
        scatter_slab(buf, b, ones)
        issue_out(buf, sem, b)

    def pair_body(j, carry):
        for b, (buf, sem) in enumerate(((buf0, sem0), (buf1, sem1))):
            k = 2 * j + b
            pltpu.make_async_copy(buf, out_hbm.at[0], sem).wait()
            scatter_slab(buf, k - 2, zeros)
            scatter_slab(buf, k, ones)
            issue_out(buf, sem, k)
        return carry

    lax.fori_loop(1, _NPW // 2, pair_body, 0)

    for buf, sem in ((buf0, sem0), (buf1, sem1)):
        pltpu.make_async_copy(buf, out_hbm.at[0], sem).wait()


def kernel(x):
    return _ten_hot(x.reshape(-1))
